# Initial kernel scaffold; baseline (speedup 1.0000x reference)
import jax
import jax.numpy as jnp
from jax.experimental import pallas as pl


def kernel(x, edge_index, edge_weight, W1, b1, W2, b2):
    raise NotImplementedError("write your pallas kernel here")


# speedup vs baseline: 13.1058x; 13.1058x over previous
import functools

import jax
import jax.numpy as jnp
from jax import lax
from jax.experimental import pallas as pl
from jax.experimental.pallas import tpu as pltpu
from jax.experimental.pallas import tpu_sc as plsc

N = 10000
E = 320000
D = 128
NC = 2
NS = 16
NW = NC * NS
NPAD = 10240
RPT = NPAD // NS

CH = 80
EPT = E // NW
NCHUNK = EPT // CH
NBUF = 5

EPT_DEG = E // NS
NCH_DEG = EPT_DEG // CH

_mesh = plsc.VectorSubcoreMesh(core_axis_name="c", subcore_axis_name="s")


def _deg_body(dst_hbm, ew_hbm, deg_hbm, dstc, ewc, zbuf, deg_sh, sem0, sem1):
    c = lax.axis_index("c")
    s = lax.axis_index("s")

    pltpu.sync_copy(dst_hbm.at[s], dstc)
    pltpu.sync_copy(ew_hbm.at[s], ewc)

    def _z(i, carry):
        zbuf[pl.ds(i * 16, 16)] = jnp.zeros((16,), jnp.float32)
        return carry
    lax.fori_loop(0, RPT // 16, _z, 0)
    pltpu.sync_copy(zbuf, deg_sh.at[pl.ds(s * RPT, RPT)])
    plsc.subcore_barrier()

    sems = (sem0, sem1)

    def _pair(p, carry):
        for b in range(2):
            ci = p * 2 + b

            @pl.when(p >= 1)
            def _():
                prev = ci - 2
                pltpu.make_async_copy(
                    ewc.at[pl.ds(prev * CH, CH)],
                    deg_sh.at[dstc.at[prev]],
                    sems[b],
                ).wait()

            pltpu.async_copy(
                ewc.at[pl.ds(ci * CH, CH)],
                deg_sh.at[dstc.at[ci]],
                sems[b],
                add=True,
            )
        return carry
    lax.fori_loop(0, NCH_DEG // 2, _pair, 0)
    for b in range(2):
        ci = NCH_DEG - 2 + b
        pltpu.make_async_copy(
            ewc.at[pl.ds(ci * CH, CH)], deg_sh.at[dstc.at[ci]], sems[b]
        ).wait()
    plsc.subcore_barrier()

    @pl.when(c == 0)
    def _():
        pltpu.sync_copy(deg_sh.at[pl.ds(s * RPT, RPT)],
                        deg_hbm.at[pl.ds(s * RPT, RPT)])


@functools.partial(jax.jit, static_argnames=())
def _deg_call(dst16, ew16):
    return pl.kernel(
        _deg_body,
        out_type=jax.ShapeDtypeStruct((NPAD,), jnp.float32),
        mesh=_mesh,
        scratch_types=[
            pltpu.VMEM((NCH_DEG, CH), jnp.int32),
            pltpu.VMEM((EPT_DEG,), jnp.float32),
            pltpu.VMEM((RPT,), jnp.float32),
            pltpu.VMEM_SHARED((NPAD,), jnp.float32),
            pltpu.SemaphoreType.DMA,
            pltpu.SemaphoreType.DMA,
        ],
    )(dst16, ew16)


D2 = D // 2


def _agg_body(hp_hbm, src_hbm, dst_hbm, ew_hbm, out_hbm,
              sidx, didx, ewc, rows, accum,
              g0, g1, g2, g3, g4, s0, s1, s2, s3, s4):
    c = lax.axis_index("c")
    s = lax.axis_index("s")
    w = c * NS + s
    gsems = (g0, g1, g2, g3, g4)
    ssems = (s0, s1, s2, s3, s4)

    pltpu.sync_copy(src_hbm.at[w], sidx)
    pltpu.sync_copy(dst_hbm.at[w], didx)
    pltpu.sync_copy(ew_hbm.at[w], ewc)

    def _zr(i, carry):
        for k in range(D2 // 16):
            rows[0, i, pl.ds(k * 16, 16)] = jnp.zeros((16,), jnp.float32)
        return carry
    lax.fori_loop(0, CH, _zr, 0)
    for blk in range(RPT // CH):
        pltpu.sync_copy(rows.at[0],
                        accum.at[pl.ds(s * RPT + blk * CH, CH)])
    plsc.subcore_barrier()

    for b in range(NBUF):
        pltpu.async_copy(hp_hbm.at[sidx.at[b]], rows.at[b], gsems[b])

    def _outer(jo, carry):
        for b in range(NBUF):
            j = jo * NBUF + b
            sf = (b + 3) % NBUF

            @pl.when(jnp.logical_and(j >= 2, j <= NCHUNK - 4))
            def _():
                pltpu.make_async_copy(
                    rows.at[sf], accum.at[didx.at[j - 2]], ssems[sf]
                ).wait()
                pltpu.async_copy(
                    hp_hbm.at[sidx.at[j + 3]], rows.at[sf], gsems[sf]
                )

            pltpu.make_async_copy(
                hp_hbm.at[sidx.at[j]], rows.at[b], gsems[b]
            ).wait()

            def _scale(g, carry2):
                sv16 = ewc[j, pl.ds(g * 16, 16)]
                for l in range(16):
                    i = g * 16 + l
                    sv = jnp.full((16,), sv16[l], jnp.float32)
                    for k in range(D2 // 16):
                        rows[b, i, pl.ds(k * 16, 16)] = (
                            rows[b, i, pl.ds(k * 16, 16)] * sv)
                return carry2
            lax.fori_loop(0, CH // 16, _scale, 0)

            pltpu.async_copy(
                rows.at[b], accum.at[didx.at[j]], ssems[b], add=True)
        return carry
    lax.fori_loop(0, NCHUNK // NBUF, _outer, 0)

    for b in range(NBUF):
        pltpu.make_async_copy(
            rows.at[b], accum.at[didx.at[NCHUNK - NBUF + b]], ssems[b]
        ).wait()
    plsc.subcore_barrier()

    for blk in range(RPT // CH):
        off = s * RPT + blk * CH
        pltpu.sync_copy(accum.at[pl.ds(off, CH)],
                        out_hbm.at[c, pl.ds(off, CH)])


def _agg_call(hp_half, src3, dst3, ew3):
    return pl.kernel(
        _agg_body,
        out_type=jax.ShapeDtypeStruct((NC, NPAD, D2), jnp.float32),
        mesh=_mesh,
        scratch_types=[
            pltpu.VMEM((NCHUNK, CH), jnp.int32),
            pltpu.VMEM((NCHUNK, CH), jnp.int32),
            pltpu.VMEM((NCHUNK, CH), jnp.float32),
            pltpu.VMEM((NBUF, CH, D2), jnp.float32),
            pltpu.VMEM_SHARED((NPAD, D2), jnp.float32),
        ] + [pltpu.SemaphoreType.DMA] * 10,
        compiler_params=pltpu.CompilerParams(use_tc_tiling_on_sc=False),
    )(hp_half, src3, dst3, ew3)


RB = 1000


def _t1_body(x_ref, w_ref, deg_ref, hp_ref):
    dinv = lax.rsqrt(deg_ref[...] + 1.0)
    h = jnp.dot(x_ref[...], w_ref[...],
                preferred_element_type=jnp.float32)
    hp_ref[...] = h * dinv


def _t1_call(x, W1, deg2):
    return pl.pallas_call(
        _t1_body,
        grid=(N // RB,),
        in_specs=[
            pl.BlockSpec((RB, D), lambda i: (i, 0)),
            pl.BlockSpec((D, D), lambda i: (0, 0)),
            pl.BlockSpec((RB, 1), lambda i: (i, 0)),
        ],
        out_specs=pl.BlockSpec((RB, D), lambda i: (i, 0)),
        out_shape=jax.ShapeDtypeStruct((N, D), jnp.float32),
    )(x, W1, deg2)


def _t2_body(pa_ref, pb_ref, hp_ref, deg_ref, b1_ref, w_ref, out_ref):
    dinv = lax.rsqrt(deg_ref[...] + 1.0)
    pa = pa_ref[...]
    pb = pb_ref[...]
    agg = jnp.concatenate([pa[0] + pa[1], pb[0] + pb[1]], axis=1)
    z = (agg + hp_ref[...]) * dinv + b1_ref[...]
    z = jnp.maximum(z, 0.0)
    out_ref[...] = jnp.dot(z, w_ref[...],
                           preferred_element_type=jnp.float32) * dinv


def _t2_call(pa, pb, hp, deg2, b1r, W2):
    return pl.pallas_call(
        _t2_body,
        grid=(N // RB,),
        in_specs=[
            pl.BlockSpec((NC, RB, D2), lambda i: (0, i, 0)),
            pl.BlockSpec((NC, RB, D2), lambda i: (0, i, 0)),
            pl.BlockSpec((RB, D), lambda i: (i, 0)),
            pl.BlockSpec((RB, 1), lambda i: (i, 0)),
            pl.BlockSpec((1, D), lambda i: (0, 0)),
            pl.BlockSpec((D, D), lambda i: (0, 0)),
        ],
        out_specs=pl.BlockSpec((RB, D), lambda i: (i, 0)),
        out_shape=jax.ShapeDtypeStruct((N, D), jnp.float32),
    )(pa, pb, hp, deg2, b1r, W2)


def _t3_body(pa_ref, pb_ref, hp_ref, deg_ref, b2_ref, out_ref):
    dinv = lax.rsqrt(deg_ref[...] + 1.0)
    pa = pa_ref[...]
    pb = pb_ref[...]
    agg = jnp.concatenate([pa[0] + pa[1], pb[0] + pb[1]], axis=1)
    out_ref[...] = (agg + hp_ref[...]) * dinv + b2_ref[...]


def _t3_call(pa, pb, hp, deg2, b2r):
    return pl.pallas_call(
        _t3_body,
        grid=(N // RB,),
        in_specs=[
            pl.BlockSpec((NC, RB, D2), lambda i: (0, i, 0)),
            pl.BlockSpec((NC, RB, D2), lambda i: (0, i, 0)),
            pl.BlockSpec((RB, D), lambda i: (i, 0)),
            pl.BlockSpec((RB, 1), lambda i: (i, 0)),
            pl.BlockSpec((1, D), lambda i: (0, 0)),
        ],
        out_specs=pl.BlockSpec((RB, D), lambda i: (i, 0)),
        out_shape=jax.ShapeDtypeStruct((N, D), jnp.float32),
    )(pa, pb, hp, deg2, b2r)


def kernel(x, edge_index, edge_weight, W1, b1, W2, b2):
    src = edge_index[0].astype(jnp.int32)
    dst = edge_index[1].astype(jnp.int32)
    ew = edge_weight.astype(jnp.float32)

    src3 = src.reshape(NW, NCHUNK, CH)
    dst3 = dst.reshape(NW, NCHUNK, CH)
    ew3 = ew.reshape(NW, NCHUNK, CH)
    dst16 = dst.reshape(NS, NCH_DEG, CH)
    ew16 = ew.reshape(NS, EPT_DEG)

    deg_pad = _deg_call(dst16, ew16)
    deg2 = deg_pad[:N].reshape(N, 1)

    hp1 = _t1_call(x, W1, deg2)
    p1a = _agg_call(lax.slice(hp1, (0, 0), (N, D2)), src3, dst3, ew3)
    p1b = _agg_call(lax.slice(hp1, (0, D2), (N, D)), src3, dst3, ew3)
    hp2 = _t2_call(p1a, p1b, hp1, deg2, b1.reshape(1, D), W2)
    p2a = _agg_call(lax.slice(hp2, (0, 0), (N, D2)), src3, dst3, ew3)
    p2b = _agg_call(lax.slice(hp2, (0, D2), (N, D)), src3, dst3, ew3)
    out = _t3_call(p2a, p2b, hp2, deg2, b2.reshape(1, D))
    return out
